# Initial kernel scaffold; baseline (speedup 1.0000x reference)
import jax
import jax.numpy as jnp
from jax.experimental import pallas as pl


def kernel(daytime, W_day, W_time, W_node):
    raise NotImplementedError("write your pallas kernel here")


# speedup vs baseline: 1.9723x; 1.9723x over previous
import jax
import jax.numpy as jnp
from jax.experimental import pallas as pl
from jax.experimental.pallas import tpu as pltpu


def _body(idx_ref, wday_ref, wtime_ref, wnode_ref, out_ref):
    b = pl.program_id(0)
    L = out_ref.shape[1]
    N, NS = wnode_ref.shape
    DS = wday_ref.shape[1]
    TS = wtime_ref.shape[1]
    node = wnode_ref[...]
    for l in range(L):
        d = idx_ref[b, l, 0]
        t = idx_ref[b, l, 1]
        day_b = jnp.broadcast_to(wday_ref[d, :][None, :], (N, DS))
        time_b = jnp.broadcast_to(wtime_ref[t, :][None, :], (N, TS))
        out_ref[0, l] = jnp.concatenate([node, day_b, time_b], axis=-1)


def kernel(daytime, W_day, W_time, W_node):
    B, L, _ = daytime.shape
    N, NS = W_node.shape
    DS = W_day.shape[1]
    TS = W_time.shape[1]
    E = NS + DS + TS

    grid_spec = pltpu.PrefetchScalarGridSpec(
        num_scalar_prefetch=1,
        grid=(B,),
        in_specs=[
            pl.BlockSpec(W_day.shape, lambda b, idx: (0, 0)),
            pl.BlockSpec(W_time.shape, lambda b, idx: (0, 0)),
            pl.BlockSpec(W_node.shape, lambda b, idx: (0, 0)),
        ],
        out_specs=pl.BlockSpec((1, L, N, E), lambda b, idx: (b, 0, 0, 0)),
    )
    return pl.pallas_call(
        _body,
        grid_spec=grid_spec,
        out_shape=jax.ShapeDtypeStruct((B, L, N, E), jnp.float32),
    )(daytime, W_day, W_time, W_node)


# speedup vs baseline: 2.1294x; 1.0797x over previous
import jax
import jax.numpy as jnp
from jax.experimental import pallas as pl
from jax.experimental.pallas import tpu as pltpu


_BB = 4


def _body(idx_ref, wday_ref, wtime_ref, wnode_ref, out_ref):
    g = pl.program_id(0)
    L = out_ref.shape[1]
    N, NS = wnode_ref.shape
    DS = wday_ref.shape[1]
    TS = wtime_ref.shape[1]
    node = wnode_ref[...]
    for bb in range(_BB):
        b = g * _BB + bb
        for l in range(L):
            d = idx_ref[b, l, 0]
            t = idx_ref[b, l, 1]
            day_b = jnp.broadcast_to(wday_ref[d, :][None, :], (N, DS))
            time_b = jnp.broadcast_to(wtime_ref[t, :][None, :], (N, TS))
            out_ref[bb, l] = jnp.concatenate([node, day_b, time_b], axis=-1)


def kernel(daytime, W_day, W_time, W_node):
    B, L, _ = daytime.shape
    N, NS = W_node.shape
    DS = W_day.shape[1]
    TS = W_time.shape[1]
    E = NS + DS + TS

    grid_spec = pltpu.PrefetchScalarGridSpec(
        num_scalar_prefetch=1,
        grid=(B // _BB,),
        in_specs=[
            pl.BlockSpec(W_day.shape, lambda b, idx: (0, 0)),
            pl.BlockSpec(W_time.shape, lambda b, idx: (0, 0)),
            pl.BlockSpec(W_node.shape, lambda b, idx: (0, 0)),
        ],
        out_specs=pl.BlockSpec((_BB, L, N, E), lambda b, idx: (b, 0, 0, 0)),
    )
    return pl.pallas_call(
        _body,
        grid_spec=grid_spec,
        out_shape=jax.ShapeDtypeStruct((B, L, N, E), jnp.float32),
    )(daytime, W_day, W_time, W_node)
